# baseline (speedup 1.0000x reference)
import functools

import jax
import jax.numpy as jnp
from jax import lax
from jax.experimental import pallas as pl
from jax.experimental.pallas import tpu as pltpu
from jax.experimental.pallas import tpu_sc as plsc

_B, _N, _V, _D = 1024, 50, 1000, 128
_BN = _B * _N
_ROWS_PER_BLK = 512
_NW = 32
_PER_W = _BN // _NW
_CH = 80
_NCH = _PER_W // _CH


def _argmax_body(x_ref, out_ref):
    blk = x_ref[...]
    m = jnp.max(blk, axis=-1, keepdims=True)
    ids = lax.broadcasted_iota(jnp.int32, blk.shape, 1)
    out_ref[...] = jnp.min(jnp.where(blk == m, ids, _V), axis=-1)


def _argmax(xf):
    return pl.pallas_call(
        _argmax_body,
        grid=(_BN // _ROWS_PER_BLK,),
        in_specs=[pl.BlockSpec((_ROWS_PER_BLK, _V), lambda i: (i, 0))],
        out_specs=pl.BlockSpec((_ROWS_PER_BLK,), lambda i: (i,)),
        out_shape=jax.ShapeDtypeStruct((_BN,), jnp.int32),
    )(xf)


_mesh = plsc.VectorSubcoreMesh(core_axis_name="c", subcore_axis_name="s")


@functools.partial(
    pl.kernel,
    out_type=jax.ShapeDtypeStruct((_BN, _D), jnp.float32),
    mesh=_mesh,
    scratch_types=[
        pltpu.VMEM((_NCH, _CH), jnp.int32),
        pltpu.VMEM((_CH, _D), jnp.float32),
        pltpu.SemaphoreType.DMA,
    ],
)
def _gather(idx_hbm, table_hbm, out_hbm, idx_v, rows_v, sem):
    wid = lax.axis_index("s") * 2 + lax.axis_index("c")
    base = wid * _PER_W
    pltpu.sync_copy(idx_hbm.at[wid], idx_v)
    for c in range(_NCH):
        pltpu.async_copy(table_hbm.at[idx_v.at[c]], rows_v, sem).wait()
        pltpu.sync_copy(rows_v, out_hbm.at[pl.ds(base + c * _CH, _CH)])


def kernel(x, table):
    xf = x.reshape(_BN, _V)
    tokens = _argmax(xf)
    out = _gather(tokens.reshape(_NW, _NCH, _CH), table)
    return out.reshape(_B, _N, _D)


# speedup vs baseline: 1.5304x; 1.5304x over previous
import functools

import jax
import jax.numpy as jnp
from jax import lax
from jax.experimental import pallas as pl
from jax.experimental.pallas import tpu as pltpu
from jax.experimental.pallas import tpu_sc as plsc

_B, _N, _V, _D = 1024, 50, 1000, 128
_BN = _B * _N
_BB = 64
_NW = 32
_PER_W = _BN // _NW
_CH = 80
_NCH = _PER_W // _CH


def _argmax_body(x_ref, out_ref):
    blk = x_ref[...]
    m = jnp.max(blk, axis=-1, keepdims=True)
    ids = lax.broadcasted_iota(jnp.int32, blk.shape, 2)
    out_ref[...] = jnp.min(jnp.where(blk == m, ids, _V), axis=-1)


def _argmax(x):
    return pl.pallas_call(
        _argmax_body,
        grid=(_B // _BB,),
        in_specs=[pl.BlockSpec((_BB, _N, _V), lambda i: (i, 0, 0))],
        out_specs=pl.BlockSpec((_BB, _N), lambda i: (i, 0)),
        out_shape=jax.ShapeDtypeStruct((_B, _N), jnp.int32),
    )(x)


_mesh = plsc.VectorSubcoreMesh(core_axis_name="c", subcore_axis_name="s")


_B_PER_W = _B // _NW


@functools.partial(
    pl.kernel,
    out_type=jax.ShapeDtypeStruct((_B, _N, _D), jnp.float32),
    mesh=_mesh,
    scratch_types=[
        pltpu.VMEM((_B_PER_W, _N), jnp.int32),
        pltpu.VMEM((2, _N, _D), jnp.float32),
        pltpu.SemaphoreType.DMA,
        pltpu.SemaphoreType.DMA,
    ],
)
def _gather(idx_hbm, table_hbm, out_hbm, idx_v, rows_v, sem0, sem1):
    wid = lax.axis_index("s") * 2 + lax.axis_index("c")
    b0 = wid * _B_PER_W
    pltpu.sync_copy(idx_hbm.at[pl.ds(b0, _B_PER_W)], idx_v)
    sems = (sem0, sem1)
    cps = [None, None]
    cps[0] = pltpu.async_copy(table_hbm.at[idx_v.at[0]], rows_v.at[0], sem0)
    for r in range(_B_PER_W):
        cur = r % 2
        if r + 1 < _B_PER_W:
            cps[1 - cur] = pltpu.async_copy(
                table_hbm.at[idx_v.at[r + 1]], rows_v.at[1 - cur], sems[1 - cur]
            )
        cps[cur].wait()
        pltpu.sync_copy(rows_v.at[cur], out_hbm.at[b0 + r])


def kernel(x, table):
    tokens = _argmax(x)
    return _gather(tokens, table)


# speedup vs baseline: 3.5290x; 2.3059x over previous
import functools

import jax
import jax.numpy as jnp
from jax import lax
from jax.experimental import pallas as pl
from jax.experimental.pallas import tpu as pltpu
from jax.experimental.pallas import tpu_sc as plsc

_B, _N, _V, _D = 1024, 50, 1000, 128
_BN = _B * _N
_BL = 512
_NW = 32
_PER_W = _BN // _NW
_CH = 80
_NCH = _PER_W // _CH


def _argmax_body(x_ref, out_ref):
    blk = x_ref[...]
    m = jnp.max(blk, axis=1, keepdims=True)
    ids = lax.broadcasted_iota(jnp.int32, blk.shape, 1)
    out_ref[...] = jnp.min(jnp.where(blk == m, ids, _V), axis=1, keepdims=True)


def _argmax(xt):
    return pl.pallas_call(
        _argmax_body,
        grid=(_N, _B // _BL),
        in_specs=[pl.BlockSpec((1, _V, _BL), lambda n, j: (n, 0, j))],
        out_specs=pl.BlockSpec((1, 1, _BL), lambda n, j: (n, 0, j)),
        out_shape=jax.ShapeDtypeStruct((_N, 1, _B), jnp.int32),
    )(xt)


_mesh = plsc.VectorSubcoreMesh(core_axis_name="c", subcore_axis_name="s")


@functools.partial(
    pl.kernel,
    out_type=jax.ShapeDtypeStruct((_BN, _D), jnp.float32),
    mesh=_mesh,
    scratch_types=[
        pltpu.VMEM((_PER_W,), jnp.int32),
        pltpu.VMEM((2, _CH, _D), jnp.float32),
        pltpu.SemaphoreType.DMA,
        pltpu.SemaphoreType.DMA,
    ],
)
def _gather(idx_hbm, table_hbm, out_hbm, idx_v, rows_v, sem0, sem1):
    wid = lax.axis_index("s") * 2 + lax.axis_index("c")
    base = wid * _PER_W
    pltpu.sync_copy(idx_hbm.at[pl.ds(base, _PER_W)], idx_v)
    sems = (sem0, sem1)
    cps = [None, None]
    cps[0] = pltpu.async_copy(
        table_hbm.at[idx_v.at[pl.ds(0, _CH)]], rows_v.at[0], sem0
    )
    for c in range(_NCH):
        cur = c % 2
        if c + 1 < _NCH:
            cps[1 - cur] = pltpu.async_copy(
                table_hbm.at[idx_v.at[pl.ds((c + 1) * _CH, _CH)]],
                rows_v.at[1 - cur],
                sems[1 - cur],
            )
        cps[cur].wait()
        pltpu.sync_copy(rows_v.at[cur], out_hbm.at[pl.ds(base + c * _CH, _CH)])


def kernel(x, table):
    xt = x.transpose(1, 2, 0)
    tokens = _argmax(xt).reshape(_BN)
    out = _gather(tokens, table)
    return out.reshape(_N, _B, _D).transpose(1, 0, 2)


# speedup vs baseline: 4.9000x; 1.3885x over previous
import functools

import jax
import jax.numpy as jnp
from jax import lax
from jax.experimental import pallas as pl
from jax.experimental.pallas import tpu as pltpu
from jax.experimental.pallas import tpu_sc as plsc

_B, _N, _V, _D = 1024, 50, 1000, 128
_BN = _B * _N
_BN_BLK = 2
_NW = 32
_PER_W = _BN // _NW
_CH = 80
_NCH = _PER_W // _CH


def _argmax_body(x_ref, out_ref):
    blk = x_ref[...]
    m = jnp.max(blk, axis=1, keepdims=True)
    ids = lax.broadcasted_iota(jnp.int32, blk.shape, 1)
    out_ref[...] = jnp.min(jnp.where(blk == m, ids, _V), axis=1, keepdims=True)


def _argmax(xt):
    return pl.pallas_call(
        _argmax_body,
        grid=(_N // _BN_BLK,),
        in_specs=[pl.BlockSpec((_BN_BLK, _V, _B), lambda n: (n, 0, 0))],
        out_specs=pl.BlockSpec((_BN_BLK, 1, _B), lambda n: (n, 0, 0)),
        out_shape=jax.ShapeDtypeStruct((_N, 1, _B), jnp.int32),
    )(xt)


_mesh = plsc.VectorSubcoreMesh(core_axis_name="c", subcore_axis_name="s")


@functools.partial(
    pl.kernel,
    out_type=jax.ShapeDtypeStruct((_BN, _D), jnp.float32),
    mesh=_mesh,
    scratch_types=[
        pltpu.VMEM((_PER_W,), jnp.int32),
        pltpu.VMEM((2, _CH, _D), jnp.float32),
        pltpu.SemaphoreType.DMA,
        pltpu.SemaphoreType.DMA,
    ],
)
def _gather(idx_hbm, table_hbm, out_hbm, idx_v, rows_v, sem0, sem1):
    wid = lax.axis_index("s") * 2 + lax.axis_index("c")
    base = wid * _PER_W
    pltpu.sync_copy(idx_hbm.at[pl.ds(base, _PER_W)], idx_v)
    sems = (sem0, sem1)
    cps = [None, None]
    cps[0] = pltpu.async_copy(
        table_hbm.at[idx_v.at[pl.ds(0, _CH)]], rows_v.at[0], sem0
    )
    for c in range(_NCH):
        cur = c % 2
        if c + 1 < _NCH:
            cps[1 - cur] = pltpu.async_copy(
                table_hbm.at[idx_v.at[pl.ds((c + 1) * _CH, _CH)]],
                rows_v.at[1 - cur],
                sems[1 - cur],
            )
        cps[cur].wait()
        pltpu.sync_copy(rows_v.at[cur], out_hbm.at[pl.ds(base + c * _CH, _CH)])


def kernel(x, table):
    xt = x.transpose(1, 2, 0)
    tokens = _argmax(xt).reshape(_BN)
    out = _gather(tokens, table)
    return out.reshape(_N, _B, _D).transpose(1, 0, 2)


# speedup vs baseline: 4.9455x; 1.0093x over previous
import functools

import jax
import jax.numpy as jnp
from jax import lax
from jax.experimental import pallas as pl
from jax.experimental.pallas import tpu as pltpu
from jax.experimental.pallas import tpu_sc as plsc

_B, _N, _V, _D = 1024, 50, 1000, 128
_BN = _B * _N
_BN_BLK = 5
_NW = 32
_PER_W = _BN // _NW
_CH = 80
_NCH = _PER_W // _CH


def _argmax_body(x_ref, out_ref):
    blk = x_ref[...]
    m = jnp.max(blk, axis=1, keepdims=True)
    ids = lax.broadcasted_iota(jnp.int32, blk.shape, 1)
    out_ref[...] = jnp.min(jnp.where(blk == m, ids, _V), axis=1, keepdims=True)


def _argmax(xt):
    return pl.pallas_call(
        _argmax_body,
        grid=(_N // _BN_BLK,),
        in_specs=[pl.BlockSpec((_BN_BLK, _V, _B), lambda n: (n, 0, 0))],
        out_specs=pl.BlockSpec((_BN_BLK, 1, _B), lambda n: (n, 0, 0)),
        out_shape=jax.ShapeDtypeStruct((_N, 1, _B), jnp.int32),
    )(xt)


_mesh = plsc.VectorSubcoreMesh(core_axis_name="c", subcore_axis_name="s")


@functools.partial(
    pl.kernel,
    out_type=jax.ShapeDtypeStruct((_BN, _D), jnp.float32),
    mesh=_mesh,
    scratch_types=[
        pltpu.VMEM((_PER_W,), jnp.int32),
        pltpu.VMEM((2, _CH, _D), jnp.float32),
        pltpu.SemaphoreType.DMA,
        pltpu.SemaphoreType.DMA,
    ],
)
def _gather(idx_hbm, table_hbm, out_hbm, idx_v, rows_v, sem0, sem1):
    wid = lax.axis_index("s") * 2 + lax.axis_index("c")
    base = wid * _PER_W
    pltpu.sync_copy(idx_hbm.at[pl.ds(base, _PER_W)], idx_v)
    sems = (sem0, sem1)
    cps = [None, None]
    cps[0] = pltpu.async_copy(
        table_hbm.at[idx_v.at[pl.ds(0, _CH)]], rows_v.at[0], sem0
    )
    for c in range(_NCH):
        cur = c % 2
        if c + 1 < _NCH:
            cps[1 - cur] = pltpu.async_copy(
                table_hbm.at[idx_v.at[pl.ds((c + 1) * _CH, _CH)]],
                rows_v.at[1 - cur],
                sems[1 - cur],
            )
        cps[cur].wait()
        pltpu.sync_copy(rows_v.at[cur], out_hbm.at[pl.ds(base + c * _CH, _CH)])


def kernel(x, table):
    xt = x.transpose(1, 2, 0)
    tokens = _argmax(xt).reshape(_BN)
    out = _gather(tokens, table)
    return out.reshape(_N, _B, _D).transpose(1, 0, 2)
